# baseline (speedup 1.0000x reference)
import functools

import jax
import jax.numpy as jnp
from jax import lax
from jax.experimental import pallas as pl
from jax.experimental.pallas import tpu as pltpu
from jax.experimental.pallas import tpu_sc as plsc

_EMB = 128
_CHUNK = 128
_NBUF = 2


@functools.lru_cache(maxsize=None)
def _build(total: int, vocab: int):
    info = plsc.get_sparse_core_info()
    nc, ns = info.num_cores, info.num_subcores
    nw = nc * ns
    assert total % (nw * _CHUNK) == 0
    n_chunks = total // (nw * _CHUNK)
    assert n_chunks % _NBUF == 0

    mesh = plsc.VectorSubcoreMesh(core_axis_name="c", subcore_axis_name="s")

    @functools.partial(
        pl.kernel,
        out_type=jax.ShapeDtypeStruct((total, _EMB), jnp.float32),
        mesh=mesh,
        scratch_types=[
            pltpu.VMEM((n_chunks * _CHUNK,), jnp.int32),
            pltpu.VMEM((_NBUF, _CHUNK, _EMB), jnp.float32),
            pltpu.SemaphoreType.DMA,
            pltpu.SemaphoreType.DMA,
        ],
    )
    def gather_kernel(idx_hbm, table_hbm, out_hbm, idx_v, rows_v, gsem, osem):
        wid = lax.axis_index("s") * nc + lax.axis_index("c")
        row0 = wid * n_chunks

        pltpu.sync_copy(idx_hbm.at[pl.ds(row0 * _CHUNK, n_chunks * _CHUNK)], idx_v)

        pltpu.async_copy(
            table_hbm.at[idx_v.at[pl.ds(0, _CHUNK)]], rows_v.at[0], gsem
        )

        def pair_body(i, _):
            for b in range(_NBUF):
                j = _NBUF * i + b
                nxt = (b + 1) % _NBUF

                @pl.when(j > 0)
                def _():
                    pltpu.make_async_copy(
                        rows_v.at[nxt],
                        out_hbm.at[pl.ds((row0 + j - 1) * _CHUNK, _CHUNK)],
                        osem,
                    ).wait()

                @pl.when(j + 1 < n_chunks)
                def _():
                    pltpu.async_copy(
                        table_hbm.at[idx_v.at[pl.ds((j + 1) * _CHUNK, _CHUNK)]],
                        rows_v.at[nxt],
                        gsem,
                    )

                pltpu.make_async_copy(
                    table_hbm.at[idx_v.at[pl.ds(j * _CHUNK, _CHUNK)]],
                    rows_v.at[b],
                    gsem,
                ).wait()
                pltpu.async_copy(
                    rows_v.at[b],
                    out_hbm.at[pl.ds((row0 + j) * _CHUNK, _CHUNK)],
                    osem,
                )
            return ()

        lax.fori_loop(0, n_chunks // _NBUF, pair_body, ())

        pltpu.make_async_copy(
            rows_v.at[(n_chunks - 1) % _NBUF],
            out_hbm.at[pl.ds((row0 + n_chunks - 1) * _CHUNK, _CHUNK)],
            osem,
        ).wait()

    return gather_kernel


def kernel(inputs, table):
    b, s, n = inputs.shape
    total = b * s * n
    idx1d = inputs.reshape(total).astype(jnp.int32)
    out = _build(total, table.shape[0])(idx1d, table)
    return out.reshape(b, s, n, table.shape[1])


# speedup vs baseline: 1.0215x; 1.0215x over previous
import functools

import jax
import jax.numpy as jnp
from jax import lax
from jax.experimental import pallas as pl
from jax.experimental.pallas import tpu as pltpu
from jax.experimental.pallas import tpu_sc as plsc

_EMB = 128
_CHUNK = 128
_NBUF = 3


@functools.lru_cache(maxsize=None)
def _build(total: int, vocab: int):
    info = plsc.get_sparse_core_info()
    nc, ns = info.num_cores, info.num_subcores
    nw = nc * ns
    assert total % (nw * _CHUNK) == 0
    n_chunks = total // (nw * _CHUNK)

    mesh = plsc.VectorSubcoreMesh(core_axis_name="c", subcore_axis_name="s")

    @functools.partial(
        pl.kernel,
        out_type=jax.ShapeDtypeStruct((total, _EMB), jnp.float32),
        mesh=mesh,
        scratch_types=[
            pltpu.VMEM((n_chunks * _CHUNK,), jnp.int32),
            pltpu.VMEM((_NBUF, _CHUNK, _EMB), jnp.float32),
            pltpu.SemaphoreType.DMA,
            pltpu.SemaphoreType.DMA,
        ],
    )
    def gather_kernel(idx_hbm, table_hbm, out_hbm, idx_v, rows_v, gsem, osem):
        wid = lax.axis_index("s") * nc + lax.axis_index("c")
        row0 = wid * n_chunks

        pltpu.sync_copy(idx_hbm.at[pl.ds(row0 * _CHUNK, n_chunks * _CHUNK)], idx_v)

        def gather_copy(j, buf):
            return pltpu.make_async_copy(
                table_hbm.at[idx_v.at[pl.ds(j * _CHUNK, _CHUNK)]],
                rows_v.at[buf],
                gsem,
            )

        def wb_copy(j, buf):
            return pltpu.make_async_copy(
                rows_v.at[buf],
                out_hbm.at[pl.ds((row0 + j) * _CHUNK, _CHUNK)],
                osem,
            )

        gather_copy(0, 0).start()

        def body(j, _):
            bj = lax.rem(j, _NBUF)
            bn = lax.rem(j + 1, _NBUF)

            @pl.when(j >= 2)
            def _():
                wb_copy(j - 2, bn).wait()

            @pl.when(j + 1 < n_chunks)
            def _():
                gather_copy(j + 1, bn).start()

            gather_copy(j, bj).wait()
            wb_copy(j, bj).start()
            return ()

        lax.fori_loop(0, n_chunks, body, ())

        wb_copy(n_chunks - 2, (n_chunks - 2) % _NBUF).wait()
        wb_copy(n_chunks - 1, (n_chunks - 1) % _NBUF).wait()

    return gather_kernel


def kernel(inputs, table):
    b, s, n = inputs.shape
    total = b * s * n
    idx1d = inputs.reshape(total).astype(jnp.int32)
    out = _build(total, table.shape[0])(idx1d, table)
    return out.reshape(b, s, n, table.shape[1])


# speedup vs baseline: 1.0221x; 1.0005x over previous
import functools

import jax
import jax.numpy as jnp
from jax import lax
from jax.experimental import pallas as pl
from jax.experimental.pallas import tpu as pltpu
from jax.experimental.pallas import tpu_sc as plsc

_EMB = 128
_CHUNK = 256
_NBUF = 3


@functools.lru_cache(maxsize=None)
def _build(total: int, vocab: int):
    info = plsc.get_sparse_core_info()
    nc, ns = info.num_cores, info.num_subcores
    nw = nc * ns
    assert total % (nw * _CHUNK) == 0
    n_chunks = total // (nw * _CHUNK)

    mesh = plsc.VectorSubcoreMesh(core_axis_name="c", subcore_axis_name="s")

    @functools.partial(
        pl.kernel,
        out_type=jax.ShapeDtypeStruct((total, _EMB), jnp.float32),
        mesh=mesh,
        scratch_types=[
            pltpu.VMEM((n_chunks * _CHUNK,), jnp.int32),
            pltpu.VMEM((_NBUF, _CHUNK, _EMB), jnp.float32),
            pltpu.SemaphoreType.DMA,
            pltpu.SemaphoreType.DMA,
        ],
    )
    def gather_kernel(idx_hbm, table_hbm, out_hbm, idx_v, rows_v, gsem, osem):
        wid = lax.axis_index("s") * nc + lax.axis_index("c")
        row0 = wid * n_chunks

        pltpu.sync_copy(idx_hbm.at[pl.ds(row0 * _CHUNK, n_chunks * _CHUNK)], idx_v)

        def gather_copy(j, buf):
            return pltpu.make_async_copy(
                table_hbm.at[idx_v.at[pl.ds(j * _CHUNK, _CHUNK)]],
                rows_v.at[buf],
                gsem,
            )

        def wb_copy(j, buf):
            return pltpu.make_async_copy(
                rows_v.at[buf],
                out_hbm.at[pl.ds((row0 + j) * _CHUNK, _CHUNK)],
                osem,
            )

        gather_copy(0, 0).start()

        def body(j, _):
            bj = lax.rem(j, _NBUF)
            bn = lax.rem(j + 1, _NBUF)

            @pl.when(j >= 2)
            def _():
                wb_copy(j - 2, bn).wait()

            @pl.when(j + 1 < n_chunks)
            def _():
                gather_copy(j + 1, bn).start()

            gather_copy(j, bj).wait()
            wb_copy(j, bj).start()
            return ()

        lax.fori_loop(0, n_chunks, body, ())

        wb_copy(n_chunks - 2, (n_chunks - 2) % _NBUF).wait()
        wb_copy(n_chunks - 1, (n_chunks - 1) % _NBUF).wait()

    return gather_kernel


def kernel(inputs, table):
    b, s, n = inputs.shape
    total = b * s * n
    idx1d = inputs.reshape(total).astype(jnp.int32)
    out = _build(total, table.shape[0])(idx1d, table)
    return out.reshape(b, s, n, table.shape[1])
